# Initial kernel scaffold; baseline (speedup 1.0000x reference)
import jax
import jax.numpy as jnp
from jax.experimental import pallas as pl


def kernel(x, edge_index, edge_attr, energy, batch, params):
    raise NotImplementedError("write your pallas kernel here")


# speedup vs baseline: 30.9671x; 30.9671x over previous
import jax
import jax.numpy as jnp
from jax import lax
from jax.experimental import pallas as pl
from jax.experimental.pallas import tpu as pltpu
from jax.experimental.pallas import tpu_sc as plsc

NN = 10000
EE = 320000
DD = 128
GB = 16
NCORES = 2
NSUB = 16
NWORK = NCORES * NSUB
EPW = EE // NWORK
WIN = 128
NWIN = EPW // WIN
TAIL = EPW - NWIN * WIN
N3 = 2500
N3PAD = 2560

_mesh = plsc.VectorSubcoreMesh(core_axis_name="c", subcore_axis_name="s")


def _seg_sizes(total, chunk=WIN):
    out = []
    while total > 0:
        out.append(min(chunk, total))
        total -= out[-1]
    return out


def _row_split(n_pad):
    for nw in range(NSUB, 0, -1):
        if n_pad % nw == 0 and (n_pad // nw) % 8 == 0:
            return nw, n_pad // nw
    raise ValueError(n_pad)


def _deg_body(row_hbm, out_hbm, idx_v, idx_t, ones_v, acc_sh):
    cid = lax.axis_index("c")
    sid = lax.axis_index("s")
    wid = sid * NCORES + cid
    base = wid * EPW
    nwr, seg = _row_split(NN)
    rbase = sid * seg

    def fill(val):
        def body(j, _):
            ones_v[j, :] = jnp.full((16,), val, jnp.float32)
            return 0
        lax.fori_loop(0, WIN, body, 0)

    fill(0.0)

    @pl.when(sid < nwr)
    def _():
        off = 0
        for sz in _seg_sizes(seg):
            pltpu.sync_copy(ones_v.at[pl.ds(0, sz)],
                            acc_sh.at[pl.ds(rbase + off, sz)])
            off += sz
    fill(1.0)
    plsc.subcore_barrier()

    def win(w, _):
        pltpu.sync_copy(row_hbm.at[pl.ds(base + w * WIN, WIN)], idx_v)
        pltpu.sync_copy(ones_v, acc_sh.at[idx_v], add=True)
        return 0
    lax.fori_loop(0, NWIN, win, 0)
    pltpu.sync_copy(row_hbm.at[pl.ds(base + NWIN * WIN, TAIL)], idx_t)
    pltpu.sync_copy(ones_v.at[pl.ds(0, TAIL)], acc_sh.at[idx_t], add=True)

    plsc.subcore_barrier()

    @pl.when(sid < nwr)
    def _():
        pltpu.sync_copy(acc_sh.at[pl.ds(rbase, seg)],
                        out_hbm.at[cid, pl.ds(rbase, seg)])


_deg_call = pl.kernel(
    _deg_body,
    out_type=jax.ShapeDtypeStruct((NCORES, NN, 16), jnp.float32),
    mesh=_mesh,
    scratch_types=[
        pltpu.VMEM((WIN,), jnp.int32),
        pltpu.VMEM((TAIL,), jnp.int32),
        pltpu.VMEM((WIN, 16), jnp.float32),
        pltpu.VMEM_SHARED((NN, 16), jnp.float32),
    ],
)


def _make_scatter(n_pad, shift):
    nwr, seg = _row_split(n_pad)

    def body(hp_hbm, row_hbm, col_hbm, ea_hbm, out_hbm,
             idx_r, idx_c, ea_v, rows_v, idx_rt, idx_ct, ea_t, rows_t,
             acc_sh, sem):
        cid = lax.axis_index("c")
        sid = lax.axis_index("s")
        wid = sid * NCORES + cid
        base = wid * EPW
        rbase = sid * seg

        def zero(j, _):
            for v in range(8):
                rows_v[j, pl.ds(v * 16, 16)] = jnp.zeros((16,), jnp.float32)
            return 0
        lax.fori_loop(0, WIN, zero, 0)

        @pl.when(sid < nwr)
        def _():
            off = 0
            for sz in _seg_sizes(seg):
                pltpu.sync_copy(rows_v.at[pl.ds(0, sz)],
                                acc_sh.at[pl.ds(rbase + off, sz)])
                off += sz
        plsc.subcore_barrier()

        def shift_idx(ref, n):
            def body(i, _):
                ref[pl.ds(i * 16, 16)] = lax.shift_right_logical(
                    ref[pl.ds(i * 16, 16)], shift)
                return 0
            lax.fori_loop(0, n // 16, body, 0)

        def scale(rows_ref, ea_ref, n):
            def body(j, _):
                spl = plsc.load_gather(ea_ref, [jnp.full((16,), j, jnp.int32)])
                for v in range(8):
                    rows_ref[j, pl.ds(v * 16, 16)] = (
                        rows_ref[j, pl.ds(v * 16, 16)] * spl)
                return 0
            lax.fori_loop(0, n, body, 0)

        def win(w, _):
            o = base + w * WIN
            pltpu.sync_copy(row_hbm.at[pl.ds(o, WIN)], idx_r)
            pltpu.sync_copy(col_hbm.at[pl.ds(o, WIN)], idx_c)
            pltpu.sync_copy(ea_hbm.at[pl.ds(o, WIN)], ea_v)
            if shift:
                shift_idx(idx_r, WIN)
                shift_idx(idx_c, WIN)
            pltpu.async_copy(hp_hbm.at[idx_r], rows_v, sem).wait()
            scale(rows_v, ea_v, WIN)
            pltpu.sync_copy(rows_v, acc_sh.at[idx_c], add=True)
            return 0
        lax.fori_loop(0, NWIN, win, 0)

        o = base + NWIN * WIN
        pltpu.sync_copy(row_hbm.at[pl.ds(o, TAIL)], idx_rt)
        pltpu.sync_copy(col_hbm.at[pl.ds(o, TAIL)], idx_ct)
        pltpu.sync_copy(ea_hbm.at[pl.ds(o, TAIL)], ea_t)
        if shift:
            shift_idx(idx_rt, TAIL)
            shift_idx(idx_ct, TAIL)
        pltpu.async_copy(hp_hbm.at[idx_rt], rows_t, sem).wait()
        scale(rows_t, ea_t, TAIL)
        pltpu.sync_copy(rows_t, acc_sh.at[idx_ct], add=True)

        plsc.subcore_barrier()

        @pl.when(sid < nwr)
        def _():
            pltpu.sync_copy(acc_sh.at[pl.ds(rbase, seg)],
                            out_hbm.at[cid, pl.ds(rbase, seg)])

    return pl.kernel(
        body,
        out_type=jax.ShapeDtypeStruct((NCORES, n_pad, DD), jnp.float32),
        mesh=_mesh,
        compiler_params=pltpu.CompilerParams(needs_layout_passes=False),
        scratch_types=[
            pltpu.VMEM((WIN,), jnp.int32),
            pltpu.VMEM((WIN,), jnp.int32),
            pltpu.VMEM((WIN,), jnp.float32),
            pltpu.VMEM((WIN, DD), jnp.float32),
            pltpu.VMEM((TAIL,), jnp.int32),
            pltpu.VMEM((TAIL,), jnp.int32),
            pltpu.VMEM((TAIL,), jnp.float32),
            pltpu.VMEM((TAIL, DD), jnp.float32),
            pltpu.VMEM_SHARED((n_pad, DD), jnp.float32),
            pltpu.SemaphoreType.DMA,
        ],
    )


_scat1_call = _make_scatter(NN, 0)
_scat3_call = _make_scatter(N3PAD, 2)


def _lrelu(t):
    return jnp.where(t >= 0, t, 0.01 * t)


def _bn(t, g, b):
    mu = jnp.mean(t, axis=0, keepdims=True)
    var = jnp.mean((t - mu) * (t - mu), axis=0, keepdims=True)
    return (t - mu) * lax.rsqrt(var + 1e-5) * g + b


def _dot(a, b):
    return lax.dot_general(a, b, (((1,), (1,)), ((), ())),
                           preferred_element_type=jnp.float32)


def _dis_from(hist2):
    deg = hist2[:, 0:1] + hist2[:, 1:2] + 1.0
    return lax.rsqrt(deg)


def _k1_body(x_ref, hist_ref, w1_ref, b1_ref, g1_ref, be1_ref, w2_ref, b2_ref,
             hp_ref):
    dis = _dis_from(hist_ref[...])
    t = _dot(x_ref[...], w1_ref[...]) + b1_ref[...]
    t = _lrelu(_bn(t, g1_ref[...], be1_ref[...]))
    h = _dot(t, w2_ref[...]) + b2_ref[...]
    hp_ref[...] = h * dis


def _k2a_body(sa_ref, sb_ref, hp_ref, x_ref, hist_ref, h2_ref):
    dis = _dis_from(hist_ref[...])
    m = dis * (sa_ref[...] + sb_ref[...] + hp_ref[...])
    h2_ref[...] = _lrelu(m + x_ref[...])


def _k2b_body(a0_ref, a1_ref, x0_ref, x1_ref, out_ref):
    hx = jnp.maximum(a0_ref[...], a1_ref[...])
    ox = jnp.maximum(x0_ref[...], x1_ref[...])
    out_ref[...] = _lrelu(hx + ox)


def _k2c_body(b0_ref, b1_ref, w1_ref, b1b_ref, g1_ref, be1_ref, w2_ref,
              b2_ref, d3_ref, hp3_ref):
    hx3 = jnp.maximum(b0_ref[...], b1_ref[...])
    t = _dot(hx3, w1_ref[...]) + b1b_ref[...]
    t = _lrelu(_bn(t, g1_ref[...], be1_ref[...]))
    h3 = _dot(t, w2_ref[...]) + b2_ref[...]
    deg3 = jnp.sum(d3_ref[...], axis=1, keepdims=True) + 1.0
    hp3_ref[...] = h3 * lax.rsqrt(deg3)


def _k3_body(sa_ref, sb_ref, hp3_ref, d3_ref, b3v_ref, energy_ref,
             we_ref, bse_ref, ge_ref, bee_ref, wl1_ref, bl1_ref, gl_ref,
             bel_ref, wl2_ref, bl2_ref, out_ref):
    deg3 = jnp.sum(d3_ref[...], axis=1, keepdims=True) + 1.0
    dis3 = lax.rsqrt(deg3)
    xo = _lrelu(dis3 * (sa_ref[...] + sb_ref[...] + hp3_ref[...]))
    b3v = b3v_ref[...]
    rows = []
    for b in range(GB):
        m = jnp.where(b3v == b, xo, -jnp.inf)
        rows.append(jnp.max(m, axis=0, keepdims=True))
    xg = jnp.concatenate(rows, axis=0)
    e = _dot(energy_ref[...], we_ref[...]) + bse_ref[...]
    e = _lrelu(_bn(e, ge_ref[...], bee_ref[...]))
    wl1 = wl1_ref[...]
    z = _dot(xg, wl1[:, :DD]) + _dot(e, wl1[:, DD:]) + bl1_ref[...]
    z = _lrelu(_bn(z, gl_ref[...], bel_ref[...]))
    out_ref[...] = _dot(wl2_ref[...], z) + bl2_ref[...]


def _tc(body, out_shape, *args):
    return pl.pallas_call(
        body, out_shape=jax.ShapeDtypeStruct(out_shape, jnp.float32))(*args)


def kernel(x, edge_index, edge_attr, energy, batch, params):
    p = params
    row = edge_index[0].astype(jnp.int32)
    col = edge_index[1].astype(jnp.int32)
    ea = edge_attr.astype(jnp.float32)
    r2 = lambda v: v.reshape(1, -1)

    deg_parts = _deg_call(row)
    hist2 = jnp.stack([deg_parts[0, :, 0], deg_parts[1, :, 0]], axis=1)
    d3grp = jnp.concatenate(
        [deg_parts[0, :, 0].reshape(N3, 4), deg_parts[1, :, 0].reshape(N3, 4)],
        axis=1)

    hp = _tc(_k1_body, (NN, DD), x, hist2,
             p['W01'], r2(p['b01']), r2(p['g01']), r2(p['be01']),
             p['W02'], r2(p['b02']))

    scat = _scat1_call(hp, row, col, ea)

    h2 = _tc(_k2a_body, (NN, DD), scat[0], scat[1], hp, x, hist2)
    hxo = _tc(_k2b_body, (NN // 2, DD),
              h2[0::2], h2[1::2], x[0::2], x[1::2])
    hp3 = _tc(_k2c_body, (N3, DD), hxo[0::2], hxo[1::2],
              p['W21'], r2(p['b21']), r2(p['g21']), r2(p['be21']),
              p['W22'], r2(p['b22']), d3grp)

    scat3 = _scat3_call(hp3, row, col, ea)

    b3v = batch[3::4].astype(jnp.int32).reshape(N3, 1)
    out = _tc(_k3_body, (1, GB),
              scat3[0, :N3], scat3[1, :N3], hp3, d3grp, b3v, energy,
              p['We'], r2(p['bE']), r2(p['ge']), r2(p['bee']),
              p['Wl1'], r2(p['bl1']), r2(p['gl']), r2(p['bel']),
              p['Wl2'], jnp.broadcast_to(p['bl2'].reshape(1, 1), (1, GB)))
    return out.T
